# baseline (speedup 1.0000x reference)
import jax
import jax.numpy as jnp
from jax import lax
from jax.experimental import pallas as pl
from jax.experimental.pallas import tpu as pltpu

T = 2048
H = 768
E = 8
RCH = 4
RR = T // RCH


def _moe_body(x_ref, rw_ref, rb_ref, b_ref, w_ref, out_hbm,
              xbf_ref, mask_ref, acc_ref, sem):
    e = pl.program_id(0)

    @pl.when(e == 0)
    def _():
        x = x_ref[...]
        xbf_ref[...] = x.astype(jnp.bfloat16)
        logits = lax.dot_general(x, rw_ref[...], (((1,), (1,)), ((), ())),
                                 preferred_element_type=jnp.float32)
        logits = logits + rb_ref[...]
        ie = lax.broadcasted_iota(jnp.int32, (T, E), 1)
        m1 = jnp.max(logits, axis=1, keepdims=True)
        e1 = jnp.min(jnp.where(logits == m1, ie, E), axis=1, keepdims=True)
        masked = jnp.where(ie == e1, -jnp.inf, logits)
        m2 = jnp.max(masked, axis=1, keepdims=True)
        e2 = jnp.min(jnp.where(masked == m2, ie, E), axis=1, keepdims=True)
        mask_ref[...] = ((ie == e1) | (ie == e2)).astype(jnp.float32)

    iota_e = lax.broadcasted_iota(jnp.int32, (T, E), 1)
    me = jnp.sum(jnp.where(iota_e == e, mask_ref[...], 0.0),
                 axis=1, keepdims=True)

    @pl.when(e < E - 1)
    def _():
        y = jnp.dot(xbf_ref[...], w_ref[0].astype(jnp.bfloat16),
                    preferred_element_type=jnp.float32)
        contrib = me * (y + b_ref[0])

        @pl.when(e == 0)
        def _():
            acc_ref[...] = contrib

        @pl.when(e > 0)
        def _():
            acc_ref[...] = acc_ref[...] + contrib

    @pl.when(e == E - 1)
    def _():
        copies = []
        for r in range(RCH):
            rs = pl.ds(r * RR, RR)
            y_r = jnp.dot(xbf_ref[rs, :], w_ref[0].astype(jnp.bfloat16),
                          preferred_element_type=jnp.float32)
            acc_ref[rs, :] = acc_ref[rs, :] + me[r * RR:(r + 1) * RR] * (
                y_r + b_ref[0])
            c = pltpu.make_async_copy(acc_ref.at[rs], out_hbm.at[rs], sem)
            c.start()
            copies.append(c)
        for c in copies:
            c.wait()


_moe = pl.pallas_call(
    _moe_body,
    grid=(E,),
    in_specs=[
        pl.BlockSpec((T, H), lambda e: (0, 0)),
        pl.BlockSpec((E, H), lambda e: (0, 0)),
        pl.BlockSpec((1, E), lambda e: (0, 0)),
        pl.BlockSpec((1, 1, H), lambda e: (e, 0, 0)),
        pl.BlockSpec((1, H, H), lambda e: (e, 0, 0)),
    ],
    out_specs=pl.BlockSpec(memory_space=pltpu.MemorySpace.HBM),
    scratch_shapes=[
        pltpu.VMEM((T, H), jnp.bfloat16),
        pltpu.VMEM((T, E), jnp.float32),
        pltpu.VMEM((T, H), jnp.float32),
        pltpu.SemaphoreType.DMA,
    ],
    out_shape=jax.ShapeDtypeStruct((T, H), jnp.float32),
)


def kernel(hidden_states, weight, bias, router_weight, router_bias):
    b, s, h = hidden_states.shape
    x = hidden_states.reshape(b * s, h)
    out = _moe(x, router_weight, router_bias.reshape(1, E),
               bias.reshape(E, 1, H), weight)
    return out.reshape(b, s, h)


# speedup vs baseline: 1.0246x; 1.0246x over previous
import jax
import jax.numpy as jnp
from jax import lax
from jax.experimental import pallas as pl
from jax.experimental.pallas import tpu as pltpu

T = 2048
H = 768
E = 8


def _moe_body(x_ref, rw_ref, rb_ref, b_ref, w_ref, out_ref, xbf_ref, mask_ref):
    e = pl.program_id(0)

    @pl.when(e == 0)
    def _():
        x = x_ref[...]
        xbf_ref[...] = x.astype(jnp.bfloat16)
        logits = lax.dot_general(x, rw_ref[...], (((1,), (1,)), ((), ())),
                                 preferred_element_type=jnp.float32)
        logits = logits + rb_ref[...]
        iota_e = lax.broadcasted_iota(jnp.int32, (T, E), 1)
        m1 = jnp.max(logits, axis=1, keepdims=True)
        e1 = jnp.min(jnp.where(logits == m1, iota_e, E), axis=1, keepdims=True)
        masked = jnp.where(iota_e == e1, -jnp.inf, logits)
        m2 = jnp.max(masked, axis=1, keepdims=True)
        e2 = jnp.min(jnp.where(masked == m2, iota_e, E), axis=1, keepdims=True)
        mask_ref[...] = ((iota_e == e1) | (iota_e == e2)).astype(jnp.float32)

    y = jnp.dot(xbf_ref[...], w_ref[0].astype(jnp.bfloat16),
                preferred_element_type=jnp.float32)
    iota_e = lax.broadcasted_iota(jnp.int32, (T, E), 1)
    me = jnp.sum(jnp.where(iota_e == e, mask_ref[...], 0.0),
                 axis=1, keepdims=True)
    contrib = me * (y + b_ref[0])

    @pl.when(e == 0)
    def _():
        out_ref[...] = contrib

    @pl.when(e > 0)
    def _():
        out_ref[...] = out_ref[...] + contrib


_moe = pl.pallas_call(
    _moe_body,
    grid=(E,),
    in_specs=[
        pl.BlockSpec((T, H), lambda e: (0, 0)),
        pl.BlockSpec((E, H), lambda e: (0, 0)),
        pl.BlockSpec((1, E), lambda e: (0, 0)),
        pl.BlockSpec((1, 1, H), lambda e: (e, 0, 0)),
        pl.BlockSpec((1, H, H), lambda e: (e, 0, 0)),
    ],
    out_specs=pl.BlockSpec((T, H), lambda e: (0, 0)),
    scratch_shapes=[
        pltpu.VMEM((T, H), jnp.bfloat16),
        pltpu.VMEM((T, E), jnp.float32),
    ],
    out_shape=jax.ShapeDtypeStruct((T, H), jnp.float32),
)


def kernel(hidden_states, weight, bias, router_weight, router_bias):
    b, s, h = hidden_states.shape
    x = hidden_states.reshape(b * s, h)
    out = _moe(x, router_weight, router_bias.reshape(1, E),
               bias.reshape(E, 1, H), weight)
    return out.reshape(b, s, h)
